# baseline (speedup 1.0000x reference)
import functools

import jax
import jax.numpy as jnp
from jax import lax
from jax.experimental import pallas as pl
from jax.experimental.pallas import tpu as pltpu
from jax.experimental.pallas import tpu_sc as plsc

N = 10000
NP = 10240
E = 320000
NTILES = 16
EP = 344064
TROWS = NP // NTILES
F = 128
CHUNK = 96
RING = 8
CHUNKW = 128
WCHUNKS = EP // (2 * NTILES * CHUNKW)
DWIN = 8
R = 1000
GRID = N // R


def _alpha1_body(x_ref, w_ref, asrc_ref, adst_ref, aso_ref, ado_ref):
    us = jnp.sum(w_ref[...] * asrc_ref[...][None, :], axis=1)
    ud = jnp.sum(w_ref[...] * adst_ref[...][None, :], axis=1)
    x = x_ref[...]
    aso_ref[...] = jnp.sum(x * us[None, :], axis=1, keepdims=True)
    ado_ref[...] = jnp.sum(x * ud[None, :], axis=1, keepdims=True)


def _alpha1(x, W, a_src, a_dst):
    return pl.pallas_call(
        _alpha1_body,
        grid=(GRID,),
        in_specs=[
            pl.BlockSpec((R, 128), lambda i: (i, 0)),
            pl.BlockSpec((128, 256), lambda i: (0, 0)),
            pl.BlockSpec((256,), lambda i: (0,)),
            pl.BlockSpec((256,), lambda i: (0,)),
        ],
        out_specs=[
            pl.BlockSpec((R, 1), lambda i: (i, 0)),
            pl.BlockSpec((R, 1), lambda i: (i, 0)),
        ],
        out_shape=[
            jax.ShapeDtypeStruct((N, 1), jnp.float32),
            jax.ShapeDtypeStruct((N, 1), jnp.float32),
        ],
    )(x, W, a_src, a_dst)


def _mm1_body(x_ref, w_ref, hs_ref):
    h = jnp.dot(x_ref[...], w_ref[...], preferred_element_type=jnp.float32)
    hs_ref[0] = h[:, :F]
    hs_ref[1] = h[:, F:]


def _mm1(x, W):
    return pl.pallas_call(
        _mm1_body,
        grid=(GRID,),
        in_specs=[
            pl.BlockSpec((R, 128), lambda i: (i, 0)),
            pl.BlockSpec((128, 256), lambda i: (0, 0)),
        ],
        out_specs=pl.BlockSpec((2, R, F), lambda i: (0, i, 0)),
        out_shape=jax.ShapeDtypeStruct((2, N, F), jnp.float32),
    )(x, W)


def _mm2_body(h_ref, w_ref, ho_ref):
    ho_ref[...] = jnp.dot(h_ref[...], w_ref[...],
                          preferred_element_type=jnp.float32)


def _mm2(h, W):
    return pl.pallas_call(
        _mm2_body,
        grid=(GRID,),
        in_specs=[
            pl.BlockSpec((R, 256), lambda i: (i, 0)),
            pl.BlockSpec((256, 128), lambda i: (0, 0)),
        ],
        out_specs=pl.BlockSpec((R, 128), lambda i: (i, 0)),
        out_shape=jax.ShapeDtypeStruct((N, 128), jnp.float32),
    )(h, W)


def _fin1_body(num_ref, den_ref, as_ref, ad_ref, b_ref, hs_ref, w2_ref,
               as2_ref, ad2_ref, h2_ref, aso_ref, ado_ref):
    a = as_ref[...] + ad_ref[...]
    wself = jnp.exp(jnp.where(a >= 0, a, 0.2 * a))
    den = den_ref[0] + den_ref[1] + wself + 1e-16
    h = jnp.concatenate([hs_ref[0], hs_ref[1]], axis=1)
    num = jnp.concatenate([num_ref[0], num_ref[1]], axis=1)
    o = (num + wself * h) / den + b_ref[...][None, :]
    h2 = jnp.maximum(o, 0.0)
    h2_ref[...] = h2
    us = jnp.sum(w2_ref[...] * as2_ref[...][None, :], axis=1)
    ud = jnp.sum(w2_ref[...] * ad2_ref[...][None, :], axis=1)
    aso_ref[...] = jnp.sum(h2 * us[None, :], axis=1, keepdims=True)
    ado_ref[...] = jnp.sum(h2 * ud[None, :], axis=1, keepdims=True)


def _fin1(num, den, a_s, a_d, b, hs, W2, a_src2, a_dst2):
    return pl.pallas_call(
        _fin1_body,
        grid=(GRID,),
        in_specs=[
            pl.BlockSpec((2, R, F), lambda i: (0, i, 0)),
            pl.BlockSpec((2, R, 1), lambda i: (0, i, 0)),
            pl.BlockSpec((R, 1), lambda i: (i, 0)),
            pl.BlockSpec((R, 1), lambda i: (i, 0)),
            pl.BlockSpec((256,), lambda i: (0,)),
            pl.BlockSpec((2, R, F), lambda i: (0, i, 0)),
            pl.BlockSpec((256, 128), lambda i: (0, 0)),
            pl.BlockSpec((128,), lambda i: (0,)),
            pl.BlockSpec((128,), lambda i: (0,)),
        ],
        out_specs=[
            pl.BlockSpec((R, 256), lambda i: (i, 0)),
            pl.BlockSpec((R, 1), lambda i: (i, 0)),
            pl.BlockSpec((R, 1), lambda i: (i, 0)),
        ],
        out_shape=[
            jax.ShapeDtypeStruct((N, 256), jnp.float32),
            jax.ShapeDtypeStruct((N, 1), jnp.float32),
            jax.ShapeDtypeStruct((N, 1), jnp.float32),
        ],
    )(num, den, a_s, a_d, b, hs, W2, a_src2, a_dst2)


def _fin2_body(num_ref, den_ref, as_ref, ad_ref, b_ref, h_ref, out_ref):
    a = as_ref[...] + ad_ref[...]
    wself = jnp.exp(jnp.where(a >= 0, a, 0.2 * a))
    den = den_ref[0] + den_ref[1] + wself + 1e-16
    num = num_ref[0] + num_ref[1]
    o = (num + wself * h_ref[...]) / den + b_ref[...][None, :]
    m = jnp.max(o, axis=1, keepdims=True)
    o = o - m
    out_ref[...] = o - jnp.log(jnp.sum(jnp.exp(o), axis=1, keepdims=True))


def _fin2(num, den, a_s, a_d, b, h):
    return pl.pallas_call(
        _fin2_body,
        grid=(GRID,),
        in_specs=[
            pl.BlockSpec((2, R, F), lambda i: (0, i, 0)),
            pl.BlockSpec((2, R, 1), lambda i: (0, i, 0)),
            pl.BlockSpec((R, 1), lambda i: (i, 0)),
            pl.BlockSpec((R, 1), lambda i: (i, 0)),
            pl.BlockSpec((128,), lambda i: (0,)),
            pl.BlockSpec((R, 128), lambda i: (i, 0)),
        ],
        out_specs=pl.BlockSpec((R, 128), lambda i: (i, 0)),
        out_shape=jax.ShapeDtypeStruct((N, 128), jnp.float32),
    )(num, den, a_s, a_d, b, h)


def _make_prepass():
    mesh = plsc.VectorSubcoreMesh(
        core_axis_name="c", subcore_axis_name="s", num_cores=2,
        num_subcores=16)

    def body(as_h, ad_h, src_h, dst_h, w_o, den_o,
             as_v, ad_v, srcv, dstv, wfull, zcol_v, den_sp, dsem):
        c = lax.axis_index("c")
        s = lax.axis_index("s")
        tsl = c * NTILES + s

        pltpu.sync_copy(as_h, as_v)
        pltpu.sync_copy(ad_h, ad_v)
        pltpu.sync_copy(src_h.at[tsl], srcv)
        pltpu.sync_copy(dst_h.at[tsl], dstv)

        zero16 = jnp.zeros((16,), jnp.float32)

        def z_col(i, carry):
            zcol_v[pl.ds(i * 16, 16)] = zero16
            return carry
        lax.fori_loop(0, TROWS // 16, z_col, 0)
        pltpu.sync_copy(zcol_v, den_sp.at[pl.ds(s * TROWS, TROWS)])

        plsc.subcore_barrier()

        iota16 = lax.broadcasted_iota(jnp.int32, (16,), 0)
        base = tsl * WCHUNKS * CHUNKW

        def chunk_body(j, carry):
            for i in range(CHUNKW // 16):
                s16 = srcv[j, pl.ds(i * 16, 16)]
                d16 = dstv[j, pl.ds(i * 16, 16)]
                av = plsc.load_gather(as_v, [s16])
                bv = plsc.load_gather(ad_v, [d16])
                e = av + bv
                e = jnp.where(e >= 0, e, jnp.float32(0.2) * e)
                w = jnp.exp(e)
                eid = base + j * CHUNKW + i * 16 + iota16
                w = jnp.where(eid < E, w, jnp.float32(0.0))
                wfull[j, pl.ds(i * 16, 16)] = w
            pltpu.async_copy(wfull.at[j], den_sp.at[dstv.at[j]], dsem,
                             add=True)

            @pl.when(j >= DWIN)
            def _():
                pltpu.make_async_copy(wfull.at[0], den_sp.at[dstv.at[0]],
                                      dsem).wait()
            return carry
        lax.fori_loop(0, WCHUNKS, chunk_body, 0)

        for _ in range(DWIN):
            pltpu.make_async_copy(wfull.at[0], den_sp.at[dstv.at[0]],
                                  dsem).wait()

        pltpu.sync_copy(wfull, w_o.at[tsl])

        plsc.subcore_barrier()
        pltpu.sync_copy(den_sp.at[pl.ds(s * TROWS, TROWS)], zcol_v)
        pltpu.sync_copy(zcol_v, den_o.at[c].at[pl.ds(s * TROWS, TROWS)])

    return pl.kernel(
        body,
        out_type=(jax.ShapeDtypeStruct((2 * NTILES, WCHUNKS, CHUNKW),
                                       jnp.float32),
                  jax.ShapeDtypeStruct((2, NP), jnp.float32)),
        mesh=mesh,
        compiler_params=pltpu.CompilerParams(needs_layout_passes=False),
        scratch_types=[
            pltpu.VMEM((NP,), jnp.float32),
            pltpu.VMEM((NP,), jnp.float32),
            pltpu.VMEM((WCHUNKS, CHUNKW), jnp.int32),
            pltpu.VMEM((WCHUNKS, CHUNKW), jnp.int32),
            pltpu.VMEM((WCHUNKS, CHUNKW), jnp.float32),
            pltpu.VMEM((TROWS,), jnp.float32),
            pltpu.VMEM_SHARED((NP,), jnp.float32),
            pltpu.SemaphoreType.DMA,
        ],
    )


def _make_mainpass(feature_split):
    nslices = NTILES if feature_split else 2 * NTILES
    chunks = EP // (nslices * CHUNK)
    rings = chunks // RING
    mesh = plsc.VectorSubcoreMesh(
        core_axis_name="c", subcore_axis_name="s", num_cores=2,
        num_subcores=16)

    def body(hs, src_h, dst_h, w_h, num_o,
             srcv, dstv, wv, rows_v, num_sp, gsem, ssem, isem):
        c = lax.axis_index("c")
        s = lax.axis_index("s")
        tsl = s if feature_split else c * NTILES + s
        hsrc = hs.at[c] if feature_split else hs

        zero16 = jnp.zeros((16,), jnp.float32)

        def z_row(r, carry):
            for k in range(F // 16):
                rows_v[0, r, pl.ds(k * 16, 16)] = zero16
            return carry
        lax.fori_loop(0, 64, z_row, 0)
        for t in range(TROWS // 64):
            pltpu.sync_copy(rows_v.at[0].at[pl.ds(0, 64)],
                            num_sp.at[pl.ds(s * TROWS + t * 64, 64)])

        pltpu.sync_copy(src_h.at[tsl].at[pl.ds(0, RING)], srcv.at[0])
        pltpu.sync_copy(dst_h.at[tsl].at[pl.ds(0, RING)], dstv.at[0])
        pltpu.sync_copy(w_h.at[tsl].at[pl.ds(0, RING)], wv.at[0])
        pltpu.async_copy(hsrc.at[srcv.at[0].at[0]], rows_v.at[0], gsem)

        plsc.subcore_barrier()

        def scatter_wait():
            pltpu.make_async_copy(rows_v.at[0],
                                  num_sp.at[dstv.at[0].at[0]], ssem).wait()

        def gather_wait():
            pltpu.make_async_copy(hsrc.at[srcv.at[0].at[0]],
                                  rows_v.at[0], gsem).wait()

        def ring_wait():
            pltpu.make_async_copy(src_h.at[tsl].at[pl.ds(0, RING)],
                                  srcv.at[0], isem).wait()
            pltpu.make_async_copy(dst_h.at[tsl].at[pl.ds(0, RING)],
                                  dstv.at[0], isem).wait()
            pltpu.make_async_copy(w_h.at[tsl].at[pl.ds(0, RING)],
                                  wv.at[0], isem).wait()

        def ring_body(r, carry):
            rb = lax.rem(r, 2)

            for jj in range(RING):
                cj = r * RING + jj
                b = lax.rem(cj, 3)

                if jj >= 2:
                    scatter_wait()
                else:
                    @pl.when(r > 0)
                    def _():
                        scatter_wait()

                if jj == 1:
                    @pl.when(r + 1 < rings)
                    def _():
                        off = (r + 1) * RING
                        pltpu.async_copy(src_h.at[tsl].at[pl.ds(off, RING)],
                                         srcv.at[1 - rb], isem)
                        pltpu.async_copy(dst_h.at[tsl].at[pl.ds(off, RING)],
                                         dstv.at[1 - rb], isem)
                        pltpu.async_copy(w_h.at[tsl].at[pl.ds(off, RING)],
                                         wv.at[1 - rb], isem)

                if jj < RING - 1:
                    pltpu.async_copy(hsrc.at[srcv.at[rb].at[jj + 1]],
                                     rows_v.at[lax.rem(cj + 1, 3)], gsem)
                else:
                    @pl.when(r + 1 < rings)
                    def _():
                        ring_wait()
                        pltpu.async_copy(hsrc.at[srcv.at[1 - rb].at[0]],
                                         rows_v.at[lax.rem(cj + 1, 3)],
                                         gsem)

                gather_wait()

                rbv = jnp.zeros((16,), jnp.int32) + rb
                jjv = jnp.full((16,), jj, jnp.int32)
                z16 = jnp.zeros((16,), jnp.int32)
                rowb = rows_v.at[b]

                def srow(rr, carry2):
                    r2 = rr * 2
                    wb0 = plsc.load_gather(wv, [rbv, jjv, z16 + r2])
                    wb1 = plsc.load_gather(wv, [rbv, jjv, z16 + (r2 + 1)])
                    for k in range(F // 16):
                        rowb[r2, pl.ds(k * 16, 16)] = (
                            rowb[r2, pl.ds(k * 16, 16)] * wb0)
                    for k in range(F // 16):
                        rowb[r2 + 1, pl.ds(k * 16, 16)] = (
                            rowb[r2 + 1, pl.ds(k * 16, 16)] * wb1)
                    return carry2
                lax.fori_loop(0, CHUNK // 2, srow, 0)

                pltpu.async_copy(rows_v.at[b],
                                 num_sp.at[dstv.at[rb].at[jj]], ssem,
                                 add=True)
            return carry

        lax.fori_loop(0, rings, ring_body, 0)

        scatter_wait()
        scatter_wait()

        plsc.subcore_barrier()

        for t in range(TROWS // 64):
            r0 = s * TROWS + t * 64
            pltpu.sync_copy(num_sp.at[pl.ds(r0, 64)],
                            rows_v.at[0].at[pl.ds(0, 64)])
            pltpu.sync_copy(rows_v.at[0].at[pl.ds(0, 64)],
                            num_o.at[c].at[pl.ds(r0, 64)])

    return pl.kernel(
        body,
        out_type=jax.ShapeDtypeStruct((2, NP, F), jnp.float32),
        mesh=mesh,
        compiler_params=pltpu.CompilerParams(needs_layout_passes=False),
        scratch_types=[
            pltpu.VMEM((2, RING, CHUNK), jnp.int32),
            pltpu.VMEM((2, RING, CHUNK), jnp.int32),
            pltpu.VMEM((2, RING, CHUNK), jnp.float32),
            pltpu.VMEM((3, CHUNK, F), jnp.float32),
            pltpu.VMEM_SHARED((NP, F), jnp.float32),
            pltpu.SemaphoreType.DMA,
            pltpu.SemaphoreType.DMA,
            pltpu.SemaphoreType.DMA,
        ],
    )


_prepass = _make_prepass()
_main_l1 = _make_mainpass(True)
_main_l2 = _make_mainpass(False)


def kernel(x, edge_index, W1, a_src1, a_dst1, b1, W2, a_src2, a_dst2, b2):
    src = edge_index[0].astype(jnp.int32)
    dst = edge_index[1].astype(jnp.int32)
    pad = jnp.zeros((EP - E,), jnp.int32)
    srcp = jnp.concatenate([src, pad])
    dstp = jnp.concatenate([dst, pad])
    src32 = srcp.reshape(2 * NTILES, WCHUNKS, CHUNKW)
    dst32 = dstp.reshape(2 * NTILES, WCHUNKS, CHUNKW)
    s16 = srcp.reshape(NTILES, EP // (NTILES * CHUNK), CHUNK)
    d16 = dstp.reshape(NTILES, EP // (NTILES * CHUNK), CHUNK)
    s32 = srcp.reshape(2 * NTILES, EP // (2 * NTILES * CHUNK), CHUNK)
    d32 = dstp.reshape(2 * NTILES, EP // (2 * NTILES * CHUNK), CHUNK)

    def padded(a):
        return jnp.pad(a.reshape(N), (0, NP - N))

    as1, ad1 = _alpha1(x, W1, a_src1, a_dst1)
    w1, den1 = _prepass(padded(as1), padded(ad1), src32, dst32)
    hs1 = _mm1(x, W1)
    w1_16 = w1.reshape(NTILES, EP // (NTILES * CHUNK), CHUNK)
    num1 = _main_l1(hs1, s16, d16, w1_16)
    h2, as2, ad2 = _fin1(num1, den1[:, :N].reshape(2, N, 1), as1, ad1, b1,
                         hs1, W2, a_src2, a_dst2)
    w2, den2 = _prepass(padded(as2), padded(ad2), src32, dst32)
    ho = _mm2(h2, W2)
    w2_32 = w2.reshape(2 * NTILES, EP // (2 * NTILES * CHUNK), CHUNK)
    num2 = _main_l2(ho, s32, d32, w2_32)
    return _fin2(num2, den2[:, :N].reshape(2, N, 1), as2, ad2, b2, ho)


# speedup vs baseline: 4.5377x; 4.5377x over previous
import functools

import jax
import jax.numpy as jnp
from jax import lax
from jax.experimental import pallas as pl
from jax.experimental.pallas import tpu as pltpu
from jax.experimental.pallas import tpu_sc as plsc

N = 10000
NP = 10240
E = 320000
NTILES = 16
EP = 344064
TROWS = NP // NTILES
F = 128
CHUNK = 96
RING = 8
CHUNKW = 128
WCHUNKS = EP // (2 * NTILES * CHUNKW)
DWIN = 8
R = 1000
GRID = N // R


def _alpha1_body(x_ref, w_ref, asrc_ref, adst_ref, aso_ref, ado_ref):
    us = jnp.sum(w_ref[...] * asrc_ref[...][None, :], axis=1)
    ud = jnp.sum(w_ref[...] * adst_ref[...][None, :], axis=1)
    x = x_ref[...]
    aso_ref[...] = jnp.sum(x * us[None, :], axis=1, keepdims=True)
    ado_ref[...] = jnp.sum(x * ud[None, :], axis=1, keepdims=True)


def _alpha1(x, W, a_src, a_dst):
    return pl.pallas_call(
        _alpha1_body,
        grid=(GRID,),
        in_specs=[
            pl.BlockSpec((R, 128), lambda i: (i, 0)),
            pl.BlockSpec((128, 256), lambda i: (0, 0)),
            pl.BlockSpec((256,), lambda i: (0,)),
            pl.BlockSpec((256,), lambda i: (0,)),
        ],
        out_specs=[
            pl.BlockSpec((R, 1), lambda i: (i, 0)),
            pl.BlockSpec((R, 1), lambda i: (i, 0)),
        ],
        out_shape=[
            jax.ShapeDtypeStruct((N, 1), jnp.float32),
            jax.ShapeDtypeStruct((N, 1), jnp.float32),
        ],
    )(x, W, a_src, a_dst)


def _mm1_body(x_ref, w_ref, hs_ref):
    h = jnp.dot(x_ref[...], w_ref[...], preferred_element_type=jnp.float32)
    hs_ref[0] = h[:, :F]
    hs_ref[1] = h[:, F:]


def _mm1(x, W):
    return pl.pallas_call(
        _mm1_body,
        grid=(GRID,),
        in_specs=[
            pl.BlockSpec((R, 128), lambda i: (i, 0)),
            pl.BlockSpec((128, 256), lambda i: (0, 0)),
        ],
        out_specs=pl.BlockSpec((2, R, F), lambda i: (0, i, 0)),
        out_shape=jax.ShapeDtypeStruct((2, N, F), jnp.float32),
    )(x, W)


def _mm2_body(h_ref, w_ref, ho_ref):
    ho_ref[...] = jnp.dot(h_ref[...], w_ref[...],
                          preferred_element_type=jnp.float32)


def _mm2(h, W):
    return pl.pallas_call(
        _mm2_body,
        grid=(GRID,),
        in_specs=[
            pl.BlockSpec((R, 256), lambda i: (i, 0)),
            pl.BlockSpec((256, 128), lambda i: (0, 0)),
        ],
        out_specs=pl.BlockSpec((R, 128), lambda i: (i, 0)),
        out_shape=jax.ShapeDtypeStruct((N, 128), jnp.float32),
    )(h, W)


def _fin1_body(num_ref, den_ref, as_ref, ad_ref, b_ref, hs_ref, w2_ref,
               as2_ref, ad2_ref, h2_ref, aso_ref, ado_ref):
    a = as_ref[...] + ad_ref[...]
    wself = jnp.exp(jnp.where(a >= 0, a, 0.2 * a))
    den = den_ref[0] + den_ref[1] + wself + 1e-16
    h = jnp.concatenate([hs_ref[0], hs_ref[1]], axis=1)
    num = jnp.concatenate([num_ref[0], num_ref[1]], axis=1)
    o = (num + wself * h) / den + b_ref[...][None, :]
    h2 = jnp.maximum(o, 0.0)
    h2_ref[...] = h2
    us = jnp.sum(w2_ref[...] * as2_ref[...][None, :], axis=1)
    ud = jnp.sum(w2_ref[...] * ad2_ref[...][None, :], axis=1)
    aso_ref[...] = jnp.sum(h2 * us[None, :], axis=1, keepdims=True)
    ado_ref[...] = jnp.sum(h2 * ud[None, :], axis=1, keepdims=True)


def _fin1(num, den, a_s, a_d, b, hs, W2, a_src2, a_dst2):
    return pl.pallas_call(
        _fin1_body,
        grid=(GRID,),
        in_specs=[
            pl.BlockSpec((2, R, F), lambda i: (0, i, 0)),
            pl.BlockSpec((2, R, 1), lambda i: (0, i, 0)),
            pl.BlockSpec((R, 1), lambda i: (i, 0)),
            pl.BlockSpec((R, 1), lambda i: (i, 0)),
            pl.BlockSpec((256,), lambda i: (0,)),
            pl.BlockSpec((2, R, F), lambda i: (0, i, 0)),
            pl.BlockSpec((256, 128), lambda i: (0, 0)),
            pl.BlockSpec((128,), lambda i: (0,)),
            pl.BlockSpec((128,), lambda i: (0,)),
        ],
        out_specs=[
            pl.BlockSpec((R, 256), lambda i: (i, 0)),
            pl.BlockSpec((R, 1), lambda i: (i, 0)),
            pl.BlockSpec((R, 1), lambda i: (i, 0)),
        ],
        out_shape=[
            jax.ShapeDtypeStruct((N, 256), jnp.float32),
            jax.ShapeDtypeStruct((N, 1), jnp.float32),
            jax.ShapeDtypeStruct((N, 1), jnp.float32),
        ],
    )(num, den, a_s, a_d, b, hs, W2, a_src2, a_dst2)


def _fin2_body(num_ref, den_ref, as_ref, ad_ref, b_ref, h_ref, out_ref):
    a = as_ref[...] + ad_ref[...]
    wself = jnp.exp(jnp.where(a >= 0, a, 0.2 * a))
    den = den_ref[0] + den_ref[1] + wself + 1e-16
    num = num_ref[0] + num_ref[1]
    o = (num + wself * h_ref[...]) / den + b_ref[...][None, :]
    m = jnp.max(o, axis=1, keepdims=True)
    o = o - m
    out_ref[...] = o - jnp.log(jnp.sum(jnp.exp(o), axis=1, keepdims=True))


def _fin2(num, den, a_s, a_d, b, h):
    return pl.pallas_call(
        _fin2_body,
        grid=(GRID,),
        in_specs=[
            pl.BlockSpec((2, R, F), lambda i: (0, i, 0)),
            pl.BlockSpec((2, R, 1), lambda i: (0, i, 0)),
            pl.BlockSpec((R, 1), lambda i: (i, 0)),
            pl.BlockSpec((R, 1), lambda i: (i, 0)),
            pl.BlockSpec((128,), lambda i: (0,)),
            pl.BlockSpec((R, 128), lambda i: (i, 0)),
        ],
        out_specs=pl.BlockSpec((R, 128), lambda i: (i, 0)),
        out_shape=jax.ShapeDtypeStruct((N, 128), jnp.float32),
    )(num, den, a_s, a_d, b, h)


def _make_prepass():
    mesh = plsc.VectorSubcoreMesh(
        core_axis_name="c", subcore_axis_name="s", num_cores=2,
        num_subcores=16)

    def body(as_h, ad_h, src_h, dst_h, w_o, den_o,
             as_v, ad_v, srcv, dstv, wfull, zcol_v, den_sp, dsem):
        c = lax.axis_index("c")
        s = lax.axis_index("s")
        tsl = c * NTILES + s

        pltpu.sync_copy(as_h, as_v)
        pltpu.sync_copy(ad_h, ad_v)
        pltpu.sync_copy(src_h.at[tsl], srcv)
        pltpu.sync_copy(dst_h.at[tsl], dstv)

        zero16 = jnp.zeros((16,), jnp.float32)

        def z_col(i, carry):
            zcol_v[pl.ds(i * 16, 16)] = zero16
            return carry
        lax.fori_loop(0, TROWS // 16, z_col, 0)
        pltpu.sync_copy(zcol_v, den_sp.at[pl.ds(s * TROWS, TROWS)])

        plsc.subcore_barrier()

        iota16 = lax.broadcasted_iota(jnp.int32, (16,), 0)
        base = tsl * WCHUNKS * CHUNKW

        def chunk_body(j, carry):
            for i in range(CHUNKW // 16):
                s16 = srcv[j, pl.ds(i * 16, 16)]
                d16 = dstv[j, pl.ds(i * 16, 16)]
                av = plsc.load_gather(as_v, [s16])
                bv = plsc.load_gather(ad_v, [d16])
                e = av + bv
                e = jnp.where(e >= 0, e, jnp.float32(0.2) * e)
                w = jnp.exp(e)
                eid = base + j * CHUNKW + i * 16 + iota16
                w = jnp.where(eid < E, w, jnp.float32(0.0))
                wfull[j, pl.ds(i * 16, 16)] = w
            pltpu.async_copy(wfull.at[j], den_sp.at[dstv.at[j]], dsem,
                             add=True)

            @pl.when(j >= DWIN)
            def _():
                pltpu.make_async_copy(wfull.at[0], den_sp.at[dstv.at[0]],
                                      dsem).wait()
            return carry
        lax.fori_loop(0, WCHUNKS, chunk_body, 0)

        for _ in range(DWIN):
            pltpu.make_async_copy(wfull.at[0], den_sp.at[dstv.at[0]],
                                  dsem).wait()

        pltpu.sync_copy(wfull, w_o.at[tsl])

        plsc.subcore_barrier()
        pltpu.sync_copy(den_sp.at[pl.ds(s * TROWS, TROWS)], zcol_v)
        pltpu.sync_copy(zcol_v, den_o.at[c].at[pl.ds(s * TROWS, TROWS)])

    return pl.kernel(
        body,
        out_type=(jax.ShapeDtypeStruct((2 * NTILES, WCHUNKS, CHUNKW),
                                       jnp.float32),
                  jax.ShapeDtypeStruct((2, NP), jnp.float32)),
        mesh=mesh,
        compiler_params=pltpu.CompilerParams(needs_layout_passes=False),
        scratch_types=[
            pltpu.VMEM((NP,), jnp.float32),
            pltpu.VMEM((NP,), jnp.float32),
            pltpu.VMEM((WCHUNKS, CHUNKW), jnp.int32),
            pltpu.VMEM((WCHUNKS, CHUNKW), jnp.int32),
            pltpu.VMEM((WCHUNKS, CHUNKW), jnp.float32),
            pltpu.VMEM((TROWS,), jnp.float32),
            pltpu.VMEM_SHARED((NP,), jnp.float32),
            pltpu.SemaphoreType.DMA,
        ],
    )


def _make_mainpass(feature_split):
    nslices = NTILES if feature_split else 2 * NTILES
    chunks = EP // (nslices * CHUNK)
    rings = chunks // RING
    mesh = plsc.VectorSubcoreMesh(
        core_axis_name="c", subcore_axis_name="s", num_cores=2,
        num_subcores=16)

    def body(hs, src_h, dst_h, w_h, num_o,
             srcv, dstv, wv, rows_v, num_sp, gsem, ssem, isem):
        c = lax.axis_index("c")
        s = lax.axis_index("s")
        tsl = s if feature_split else c * NTILES + s
        hsrc = hs.at[c] if feature_split else hs

        zero16 = jnp.zeros((16,), jnp.float32)

        def z_row(r, carry):
            for k in range(F // 16):
                rows_v[0, r, pl.ds(k * 16, 16)] = zero16
            return carry
        lax.fori_loop(0, 64, z_row, 0)
        for t in range(TROWS // 64):
            pltpu.sync_copy(rows_v.at[0].at[pl.ds(0, 64)],
                            num_sp.at[pl.ds(s * TROWS + t * 64, 64)])

        pltpu.sync_copy(src_h.at[tsl].at[pl.ds(0, RING)], srcv.at[0])
        pltpu.sync_copy(dst_h.at[tsl].at[pl.ds(0, RING)], dstv.at[0])
        pltpu.sync_copy(w_h.at[tsl].at[pl.ds(0, RING)], wv.at[0])
        pltpu.async_copy(hsrc.at[srcv.at[0].at[0]], rows_v.at[0], gsem)

        plsc.subcore_barrier()

        def scatter_wait():
            pltpu.make_async_copy(rows_v.at[0],
                                  num_sp.at[dstv.at[0].at[0]], ssem).wait()

        def gather_wait():
            pltpu.make_async_copy(hsrc.at[srcv.at[0].at[0]],
                                  rows_v.at[0], gsem).wait()

        def ring_wait():
            pltpu.make_async_copy(src_h.at[tsl].at[pl.ds(0, RING)],
                                  srcv.at[0], isem).wait()
            pltpu.make_async_copy(dst_h.at[tsl].at[pl.ds(0, RING)],
                                  dstv.at[0], isem).wait()
            pltpu.make_async_copy(w_h.at[tsl].at[pl.ds(0, RING)],
                                  wv.at[0], isem).wait()

        def ring_body(r, carry):
            rb = lax.rem(r, 2)

            for jj in range(RING):
                cj = r * RING + jj
                b = lax.rem(cj, 3)

                if jj >= 2:
                    scatter_wait()
                else:
                    @pl.when(r > 0)
                    def _():
                        scatter_wait()

                if jj == 1:
                    @pl.when(r + 1 < rings)
                    def _():
                        off = (r + 1) * RING
                        pltpu.async_copy(src_h.at[tsl].at[pl.ds(off, RING)],
                                         srcv.at[1 - rb], isem)
                        pltpu.async_copy(dst_h.at[tsl].at[pl.ds(off, RING)],
                                         dstv.at[1 - rb], isem)
                        pltpu.async_copy(w_h.at[tsl].at[pl.ds(off, RING)],
                                         wv.at[1 - rb], isem)

                if jj < RING - 1:
                    pltpu.async_copy(hsrc.at[srcv.at[rb].at[jj + 1]],
                                     rows_v.at[lax.rem(cj + 1, 3)], gsem)
                else:
                    @pl.when(r + 1 < rings)
                    def _():
                        ring_wait()
                        pltpu.async_copy(hsrc.at[srcv.at[1 - rb].at[0]],
                                         rows_v.at[lax.rem(cj + 1, 3)],
                                         gsem)

                gather_wait()

                rbv = jnp.zeros((16,), jnp.int32) + rb
                jjv = jnp.full((16,), jj, jnp.int32)
                z16 = jnp.zeros((16,), jnp.int32)
                rowb = rows_v.at[b]

                def srow(rr, carry2):
                    r2 = rr * 2
                    wb0 = plsc.load_gather(wv, [rbv, jjv, z16 + r2])
                    wb1 = plsc.load_gather(wv, [rbv, jjv, z16 + (r2 + 1)])
                    for k in range(F // 16):
                        rowb[r2, pl.ds(k * 16, 16)] = (
                            rowb[r2, pl.ds(k * 16, 16)] * wb0)
                    for k in range(F // 16):
                        rowb[r2 + 1, pl.ds(k * 16, 16)] = (
                            rowb[r2 + 1, pl.ds(k * 16, 16)] * wb1)
                    return carry2
                lax.fori_loop(0, CHUNK // 2, srow, 0)

                pltpu.async_copy(rows_v.at[b],
                                 num_sp.at[dstv.at[rb].at[jj]], ssem,
                                 add=True)
            return carry

        lax.fori_loop(0, rings, ring_body, 0)

        scatter_wait()
        scatter_wait()

        plsc.subcore_barrier()

        for t in range(TROWS // 64):
            r0 = s * TROWS + t * 64
            pltpu.sync_copy(num_sp.at[pl.ds(r0, 64)],
                            rows_v.at[0].at[pl.ds(0, 64)])
            pltpu.sync_copy(rows_v.at[0].at[pl.ds(0, 64)],
                            num_o.at[c].at[pl.ds(r0, 64)])

    return pl.kernel(
        body,
        out_type=jax.ShapeDtypeStruct((2, NP, F), jnp.float32),
        mesh=mesh,
        compiler_params=pltpu.CompilerParams(needs_layout_passes=False),
        scratch_types=[
            pltpu.VMEM((2, RING, CHUNK), jnp.int32),
            pltpu.VMEM((2, RING, CHUNK), jnp.int32),
            pltpu.VMEM((2, RING, CHUNK), jnp.float32),
            pltpu.VMEM((3, CHUNK, F), jnp.float32),
            pltpu.VMEM_SHARED((NP, F), jnp.float32),
            pltpu.SemaphoreType.DMA,
            pltpu.SemaphoreType.DMA,
            pltpu.SemaphoreType.DMA,
        ],
    )


_prepass = _make_prepass()
_main_l1 = _make_mainpass(True)
_main_l2 = _make_mainpass(False)


def kernel(x, edge_index, W1, a_src1, a_dst1, b1, W2, a_src2, a_dst2, b2):
    src = edge_index[0].astype(jnp.int32)
    dst = edge_index[1].astype(jnp.int32)
    padi = jnp.arange(EP - E, dtype=jnp.int32)
    srcp = jnp.concatenate([src, padi % N])
    dstp = jnp.concatenate([dst, N + padi % (NP - N)])
    src32 = srcp.reshape(2 * NTILES, WCHUNKS, CHUNKW)
    dst32 = dstp.reshape(2 * NTILES, WCHUNKS, CHUNKW)
    s16 = srcp.reshape(NTILES, EP // (NTILES * CHUNK), CHUNK)
    d16 = dstp.reshape(NTILES, EP // (NTILES * CHUNK), CHUNK)
    s32 = srcp.reshape(2 * NTILES, EP // (2 * NTILES * CHUNK), CHUNK)
    d32 = dstp.reshape(2 * NTILES, EP // (2 * NTILES * CHUNK), CHUNK)

    def padded(a):
        return jnp.pad(a.reshape(N), (0, NP - N))

    as1, ad1 = _alpha1(x, W1, a_src1, a_dst1)
    w1, den1 = _prepass(padded(as1), padded(ad1), src32, dst32)
    hs1 = _mm1(x, W1)
    w1_16 = w1.reshape(NTILES, EP // (NTILES * CHUNK), CHUNK)
    num1 = _main_l1(hs1, s16, d16, w1_16)
    h2, as2, ad2 = _fin1(num1, den1[:, :N].reshape(2, N, 1), as1, ad1, b1,
                         hs1, W2, a_src2, a_dst2)
    w2, den2 = _prepass(padded(as2), padded(ad2), src32, dst32)
    ho = _mm2(h2, W2)
    w2_32 = w2.reshape(2 * NTILES, EP // (2 * NTILES * CHUNK), CHUNK)
    num2 = _main_l2(ho, s32, d32, w2_32)
    return _fin2(num2, den2[:, :N].reshape(2, N, 1), as2, ad2, b2, ho)
